# baseline (speedup 1.0000x reference)
import jax
import jax.numpy as jnp
from jax.experimental import pallas as pl
from jax.experimental.pallas import tpu as pltpu
from jax.experimental.pallas import tpu_sc as plsc

D_MODEL = 64
SCALE = 8.0
W = 128
LANES = 16


def kernel(x, table):
    n = x.shape[0] * x.shape[1]
    idx = x.reshape(1, n)
    mesh = plsc.VectorSubcoreMesh(core_axis_name="c", subcore_axis_name="s")

    @pl.kernel(
        out_type=jax.ShapeDtypeStruct((n, D_MODEL), jnp.float32),
        mesh=mesh,
        compiler_params=pltpu.CompilerParams(use_tc_tiling_on_sc=False),
    )
    def gather_kernel(table_hbm, idx_hbm, out_hbm):
        def body(i_vmem, o_vmem):
            pltpu.sync_copy(table_hbm.at[i_vmem.at[0]], o_vmem)

            @pl.loop(0, W)
            def _(r):
                for c in range(0, D_MODEL, LANES):
                    slc = (pl.ds(r, 1), pl.ds(c, LANES))
                    o_vmem.at[*slc][...] = o_vmem.at[*slc][...] * SCALE

        pltpu.emit_pipeline(
            body,
            grid=(n // W,),
            in_specs=[pl.BlockSpec((1, W), lambda i: (0, i))],
            out_specs=[pl.BlockSpec((W, D_MODEL), lambda i: (i, 0))],
            core_axis_name=("c", "s"),
            dimension_semantics=(pltpu.PARALLEL,),
        )(idx_hbm, out_hbm)

    out = gather_kernel(table, idx)
    return out.reshape(x.shape[0], x.shape[1], D_MODEL)


# speedup vs baseline: 1.4944x; 1.4944x over previous
import jax
import jax.numpy as jnp
from jax import lax
from jax.experimental import pallas as pl
from jax.experimental.pallas import tpu as pltpu
from jax.experimental.pallas import tpu_sc as plsc

D_MODEL = 64
SCALE = 8.0
CH = 128
NBUF = 4
LANES = 16
NC, NS = 2, 16
NW = NC * NS


def kernel(x, table):
    n = x.shape[0] * x.shape[1]
    idx = x.reshape(n)
    n_per_w = n // NW
    n_ch = n_per_w // CH
    mesh = plsc.VectorSubcoreMesh(core_axis_name="c", subcore_axis_name="s")

    @pl.kernel(
        out_type=jax.ShapeDtypeStruct((n, D_MODEL), jnp.float32),
        mesh=mesh,
        scratch_types=[
            pltpu.VMEM((n_per_w,), jnp.int32),
            pltpu.VMEM((NBUF, CH, D_MODEL), jnp.float32),
            pltpu.VMEM((NBUF, CH, D_MODEL), jnp.float32),
            pltpu.SemaphoreType.DMA((NBUF,)),
            pltpu.SemaphoreType.DMA((NBUF,)),
        ],
        compiler_params=pltpu.CompilerParams(use_tc_tiling_on_sc=False),
    )
    def gather_kernel(table_hbm, idx_hbm, out_hbm, idx_v, rows_g, rows_o,
                      gsem, osem):
        wid = lax.axis_index("s") * NC + lax.axis_index("c")
        base = wid * n_per_w
        pltpu.sync_copy(idx_hbm.at[pl.ds(base, n_per_w)], idx_v)

        def start_gather(c, b):
            pltpu.make_async_copy(
                table_hbm.at[idx_v.at[pl.ds(c * CH, CH)]],
                rows_g.at[b], gsem.at[b]).start()

        def wait_gather(b):
            pltpu.make_async_copy(
                table_hbm.at[idx_v.at[pl.ds(0, CH)]],
                rows_g.at[b], gsem.at[b]).wait()

        def start_out(c, b):
            pltpu.make_async_copy(
                rows_o.at[b],
                out_hbm.at[pl.ds(base + c * CH, CH)], osem.at[b]).start()

        def wait_out(b):
            pltpu.make_async_copy(
                rows_o.at[b],
                out_hbm.at[pl.ds(base, CH)], osem.at[b]).wait()

        def scale_chunk(b):
            src = rows_g.at[b]
            dst = rows_o.at[b]

            @pl.loop(0, CH, step=4)
            def _(r):
                for dr in range(4):
                    for col in range(0, D_MODEL, LANES):
                        slc = (pl.ds(r + dr, 1), pl.ds(col, LANES))
                        dst.at[*slc][...] = src.at[*slc][...] * SCALE

        for b in range(NBUF):
            start_gather(b, b)
        for b in range(NBUF):
            wait_gather(b)
            scale_chunk(b)
            start_out(b, b)
            start_gather(b + NBUF, b)

        @pl.loop(NBUF, n_ch - NBUF, step=NBUF)
        def _(g):
            for b in range(NBUF):
                c = g + b
                wait_gather(b)
                wait_out(b)
                scale_chunk(b)
                start_out(c, b)
                start_gather(c + NBUF, b)

        for b in range(NBUF):
            c = n_ch - NBUF + b
            wait_gather(b)
            wait_out(b)
            scale_chunk(b)
            start_out(c, b)
        for b in range(NBUF):
            wait_out(b)

    out = gather_kernel(table, idx)
    return out.reshape(x.shape[0], x.shape[1], D_MODEL)
